# Initial kernel scaffold; baseline (speedup 1.0000x reference)
import jax
import jax.numpy as jnp
from jax.experimental import pallas as pl


def kernel(x, Wg, bg, We, be):
    raise NotImplementedError("write your pallas kernel here")


# speedup vs baseline: 1.2069x; 1.2069x over previous
import jax
import jax.numpy as jnp
from jax.experimental import pallas as pl
from jax.experimental.pallas import tpu as pltpu

_TS = 512


def _gate_counts_body(x_ref, wg_ref, bg_ref, out_ref):
    s, e = x_ref.shape[1], wg_ref.shape[0]
    logits = jax.lax.dot_general(
        x_ref[0], wg_ref[...],
        dimension_numbers=(((1,), (1,)), ((), ())),
        preferred_element_type=jnp.float32,
        precision=jax.lax.Precision.HIGHEST,
    )
    logits = logits + bg_ref[0][None, :]
    m = jnp.max(logits, axis=1, keepdims=True)
    iota = jax.lax.broadcasted_iota(jnp.int32, (s, e), 1)
    idx = jnp.min(jnp.where(logits >= m, iota, e), axis=1, keepdims=True)
    onehot = (idx == iota).astype(jnp.float32)
    out_ref[0, 0, :] = jnp.sum(onehot, axis=0)


def _moe_body(counts_ref, x_ref, we_ref, be_ref, out_ref):
    b = pl.program_id(0)
    e = pl.program_id(2)
    n_e = pl.num_programs(2)
    inv_s = 1.0 / (x_ref.shape[1] * pl.num_programs(1))
    wt = counts_ref[b, e] * inv_s
    z = jnp.dot(x_ref[0], we_ref[0], preferred_element_type=jnp.float32)
    z = z + be_ref[e, :][None, :]
    contrib = wt * jnp.maximum(z, 0.0)

    @pl.when(e == 0)
    def _():
        out_ref[0] = contrib

    @pl.when(e > 0)
    def _():
        out_ref[0] += contrib


def kernel(x, Wg, bg, We, be):
    B, S, D = x.shape
    E = Wg.shape[0]

    counts = pl.pallas_call(
        _gate_counts_body,
        grid=(B,),
        in_specs=[
            pl.BlockSpec((1, S, D), lambda b: (b, 0, 0)),
            pl.BlockSpec((E, D), lambda b: (0, 0)),
            pl.BlockSpec((1, E), lambda b: (0, 0)),
        ],
        out_specs=pl.BlockSpec((1, 1, E), lambda b: (b, 0, 0)),
        out_shape=jax.ShapeDtypeStruct((B, 1, E), jnp.float32),
    )(x, Wg, bg.reshape(1, E))
    counts = counts.reshape(B, E)

    xb = x.astype(jnp.bfloat16)
    we_t = jnp.swapaxes(We, 1, 2).astype(jnp.bfloat16)

    s_tiles = S // _TS
    out = pl.pallas_call(
        _moe_body,
        grid=(B, s_tiles, E),
        in_specs=[
            pl.BlockSpec(memory_space=pltpu.SMEM),
            pl.BlockSpec((1, _TS, D), lambda b, s, e: (b, s, 0)),
            pl.BlockSpec((1, D, D), lambda b, s, e: (e, 0, 0)),
            pl.BlockSpec((E, D), lambda b, s, e: (0, 0)),
        ],
        out_specs=pl.BlockSpec((1, _TS, D), lambda b, s, e: (b, s, 0)),
        out_shape=jax.ShapeDtypeStruct((B, S, D), jnp.float32),
        compiler_params=pltpu.CompilerParams(
            dimension_semantics=("parallel", "parallel", "arbitrary"),
        ),
    )(counts, xb, we_t, be)
    return out


# speedup vs baseline: 1.6566x; 1.3726x over previous
import jax
import jax.numpy as jnp
from jax.experimental import pallas as pl
from jax.experimental.pallas import tpu as pltpu


def _moe_body(x_ref, wg_ref, bg_ref, we_ref, be_ref, out_ref, w_ref, xb_ref):
    e = pl.program_id(1)
    n_e = pl.num_programs(1)
    s, d = x_ref.shape[1], x_ref.shape[2]

    @pl.when(e == 0)
    def _gate():
        xf = x_ref[0]
        logits = jax.lax.dot_general(
            xf, wg_ref[...],
            dimension_numbers=(((1,), (1,)), ((), ())),
            preferred_element_type=jnp.float32,
            precision=jax.lax.Precision.HIGHEST,
        )
        logits = logits + bg_ref[0][None, :]
        m = jnp.max(logits, axis=1, keepdims=True)
        iota = jax.lax.broadcasted_iota(jnp.int32, (s, n_e), 1)
        idx = jnp.min(jnp.where(logits >= m, iota, n_e), axis=1, keepdims=True)
        onehot = (idx == iota).astype(jnp.float32)
        counts = jnp.sum(onehot, axis=0)
        for j in range(n_e):
            w_ref[j] = counts[j] * (1.0 / s)
        xb_ref[...] = xf.astype(jnp.bfloat16)

    wt = w_ref[e]
    z = jnp.dot(xb_ref[...], we_ref[0], preferred_element_type=jnp.float32)
    contrib = wt * jnp.maximum(z + be_ref[e, :][None, :], 0.0)

    @pl.when(e == 0)
    def _():
        out_ref[0] = contrib

    @pl.when(e > 0)
    def _():
        out_ref[0] += contrib


def kernel(x, Wg, bg, We, be):
    B, S, D = x.shape
    E = Wg.shape[0]

    we_t = jnp.swapaxes(We, 1, 2).astype(jnp.bfloat16)

    out = pl.pallas_call(
        _moe_body,
        grid=(B, E),
        in_specs=[
            pl.BlockSpec((1, S, D), lambda b, e: (b, 0, 0)),
            pl.BlockSpec((E, D), lambda b, e: (0, 0)),
            pl.BlockSpec((1, E), lambda b, e: (0, 0)),
            pl.BlockSpec((1, D, D), lambda b, e: (e, 0, 0)),
            pl.BlockSpec((E, D), lambda b, e: (0, 0)),
        ],
        out_specs=pl.BlockSpec((1, S, D), lambda b, e: (b, 0, 0)),
        out_shape=jax.ShapeDtypeStruct((B, S, D), jnp.float32),
        scratch_shapes=[
            pltpu.SMEM((E,), jnp.float32),
            pltpu.VMEM((S, D), jnp.bfloat16),
        ],
        compiler_params=pltpu.CompilerParams(
            dimension_semantics=("arbitrary", "arbitrary"),
        ),
    )(x, Wg, bg.reshape(1, E), we_t, be)
    return out


# speedup vs baseline: 2.0689x; 1.2489x over previous
import jax
import jax.numpy as jnp
from jax.experimental import pallas as pl
from jax.experimental.pallas import tpu as pltpu

_TS = 256
_TF = 256


def _moe_body(x_ref, wg_ref, bg_ref, we_ref, be_ref, out_ref, w_ref, xb_ref):
    s, d = x_ref.shape[1], x_ref.shape[2]
    n_e = wg_ref.shape[0]

    xf = x_ref[0]
    dn = (((1,), (1,)), ((), ()))
    xh = xf.astype(jnp.bfloat16)
    xl = (xf - xh.astype(jnp.float32)).astype(jnp.bfloat16)
    wg = wg_ref[...]
    wh = wg.astype(jnp.bfloat16)
    wl = (wg - wh.astype(jnp.float32)).astype(jnp.bfloat16)
    logits = (
        jax.lax.dot_general(xh, wh, dn, preferred_element_type=jnp.float32)
        + jax.lax.dot_general(xh, wl, dn, preferred_element_type=jnp.float32)
        + jax.lax.dot_general(xl, wh, dn, preferred_element_type=jnp.float32)
    )
    logits = logits + bg_ref[0][None, :]
    m = jnp.max(logits, axis=1, keepdims=True)
    iota = jax.lax.broadcasted_iota(jnp.int32, (s, n_e), 1)
    idx = jnp.min(jnp.where(logits >= m, iota, n_e), axis=1, keepdims=True)
    onehot = (idx == iota).astype(jnp.float32)
    counts = jnp.sum(onehot, axis=0)
    for j in range(n_e):
        w_ref[j] = counts[j] * (1.0 / s)
    xb_ref[...] = xh

    for st in range(s // _TS):
        xs = xb_ref[pl.ds(st * _TS, _TS), :]
        for ft in range(d // _TF):
            acc = None
            for e in range(n_e):
                z = jnp.dot(
                    xs,
                    we_ref[e, :, pl.ds(ft * _TF, _TF)],
                    preferred_element_type=jnp.float32,
                )
                zb = z + be_ref[e, pl.ds(ft * _TF, _TF)][None, :]
                c = w_ref[e] * jnp.maximum(zb, 0.0)
                acc = c if acc is None else acc + c
            out_ref[0, pl.ds(st * _TS, _TS), pl.ds(ft * _TF, _TF)] = acc


def kernel(x, Wg, bg, We, be):
    B, S, D = x.shape
    E = Wg.shape[0]

    we_t = jnp.swapaxes(We, 1, 2).astype(jnp.bfloat16)

    out = pl.pallas_call(
        _moe_body,
        grid=(B,),
        in_specs=[
            pl.BlockSpec((1, S, D), lambda b: (b, 0, 0)),
            pl.BlockSpec((E, D), lambda b: (0, 0)),
            pl.BlockSpec((1, E), lambda b: (0, 0)),
            pl.BlockSpec((E, D, D), lambda b: (0, 0, 0)),
            pl.BlockSpec((E, D), lambda b: (0, 0)),
        ],
        out_specs=pl.BlockSpec((1, S, D), lambda b: (b, 0, 0)),
        out_shape=jax.ShapeDtypeStruct((B, S, D), jnp.float32),
        scratch_shapes=[
            pltpu.SMEM((E,), jnp.float32),
            pltpu.VMEM((S, D), jnp.bfloat16),
        ],
        compiler_params=pltpu.CompilerParams(
            dimension_semantics=("arbitrary",),
        ),
    )(x, Wg, bg.reshape(1, E), we_t, be)
    return out
